# baseline (speedup 1.0000x reference)
import functools

import jax
import jax.numpy as jnp
from jax.experimental import pallas as pl
from jax.experimental.pallas import tpu as pltpu


def _round_up(a: int, b: int) -> int:
    return (a + b - 1) // b * b


def _make_fc_kernel(tm: int, k: int, h: int, g: int):
    g_half = g // 2 if g % 2 == 0 else g

    def _fc_kernel(x_hbm, w_hbm, b_ref, o_ref, xbuf, wbuf, xsems, wsems):
        i = pl.program_id(0)
        slot = jax.lax.rem(i, 2)

        def x_cp(step, sl, j):
            return pltpu.make_async_copy(
                x_hbm.at[pl.ds(step * tm, tm), j, :],
                xbuf.at[sl, j],
                xsems.at[sl, j])

        def w_cp(j):
            return pltpu.make_async_copy(
                w_hbm.at[pl.ds(j * h, h), :],
                wbuf.at[pl.ds(j * h, h), :],
                wsems.at[j])

        is_first = (i == 0) | (i == g_half)

        @pl.when(is_first)
        def _warmup():
            for j in range(k):
                x_cp(i, slot, j).start()
                w_cp(j).start()

        nxt_ok = (i + 1 < g) & (i + 1 != g_half)

        @pl.when(nxt_ok)
        def _prefetch():
            for j in range(k):
                x_cp(i + 1, 1 - slot, j).start()

        y = None
        for j in range(k):
            x_cp(i, slot, j).wait()

            @pl.when(is_first)
            def _wait_w():
                w_cp(j).wait()

            d = jnp.dot(xbuf[slot, j], wbuf[j * h:(j + 1) * h, :],
                        preferred_element_type=jnp.float32)
            y = d if y is None else y + d
        o_ref[...] = jnp.maximum(y + b_ref[...], 0.0).astype(o_ref.dtype)

    return _fc_kernel


@functools.partial(jax.jit, static_argnames=("tm",))
def _fc_apply(x, w_t, b_row, tm: int):
    n, k, h = x.shape
    kin = w_t.shape[0]
    out_dtype = x.dtype

    n_pad = _round_up(n, tm)
    if n_pad != n:
        x = jnp.pad(x, ((0, n_pad - n), (0, 0), (0, 0)))

    g = n_pad // tm

    cost = pl.CostEstimate(
        flops=2 * n_pad * kin * h,
        transcendentals=0,
        bytes_accessed=(n_pad * kin * 4 + 2 * kin * h * 4 + h * 4
                        + n_pad * h * 4),
    )

    out = pl.pallas_call(
        _make_fc_kernel(tm, k, h, g),
        out_shape=jax.ShapeDtypeStruct((n_pad, h), out_dtype),
        grid=(g,),
        in_specs=[pl.BlockSpec(memory_space=pltpu.MemorySpace.HBM),
                  pl.BlockSpec(memory_space=pltpu.MemorySpace.HBM),
                  pl.BlockSpec((1, h), lambda i: (0, 0))],
        out_specs=pl.BlockSpec((tm, h), lambda i: (i, 0)),
        scratch_shapes=[pltpu.VMEM((2, k, tm, h), jnp.float32),
                        pltpu.VMEM((kin, h), jnp.float32),
                        pltpu.SemaphoreType.DMA((2, k)),
                        pltpu.SemaphoreType.DMA((k,))],
        compiler_params=pltpu.CompilerParams(
            dimension_semantics=("parallel",),
            vmem_limit_bytes=60 * 1024 * 1024,
        ),
        cost_estimate=cost,
    )(x, w_t, b_row)

    if n_pad != n:
        out = out[:n]
    return out


def kernel(x, w_t, b_row):
    n = x.shape[0]
    tm = 256 if n >= 1024 else max(8, _round_up(n // 2, 8))
    return _fc_apply(x, w_t, b_row, tm)


# speedup vs baseline: 1.0940x; 1.0940x over previous
import functools

import jax
import jax.numpy as jnp
from jax.experimental import pallas as pl
from jax.experimental.pallas import tpu as pltpu


def _round_up(a: int, b: int) -> int:
    return (a + b - 1) // b * b


def _make_fc_kernel(tm: int, k: int, h: int, g: int):
    g_half = g // 2 if g % 2 == 0 else g

    def _fc_kernel(x_hbm, w_hbm, b_ref, o_ref, xbuf, wbuf, xsems, wsems):
        i = pl.program_id(0)
        slot = jax.lax.rem(i, 2)

        def x_cp(step, sl, j):
            return pltpu.make_async_copy(
                x_hbm.at[pl.ds(step * tm, tm), j, :],
                xbuf.at[sl, j],
                xsems.at[sl, j])

        def w_cp(j):
            return pltpu.make_async_copy(
                w_hbm.at[pl.ds(j * h, h), :],
                wbuf.at[pl.ds(j * h, h), :],
                wsems.at[j])

        is_first = (i == 0) | (i == g_half)

        @pl.when(is_first)
        def _warmup():
            for j in range(k):
                x_cp(i, slot, j).start()
                w_cp(j).start()

        nxt_ok = (i + 1 < g) & (i + 1 != g_half)

        @pl.when(nxt_ok)
        def _prefetch():
            for j in range(k):
                x_cp(i + 1, 1 - slot, j).start()

        y = None
        for j in range(k):
            x_cp(i, slot, j).wait()

            @pl.when(is_first)
            def _wait_w():
                w_cp(j).wait()

            d = jnp.dot(xbuf[slot, j], wbuf[j * h:(j + 1) * h, :],
                        preferred_element_type=jnp.float32)
            y = d if y is None else y + d
        o_ref[...] = jnp.maximum(y + b_ref[...], 0.0).astype(o_ref.dtype)

    return _fc_kernel


@functools.partial(jax.jit, static_argnames=("tm",))
def _fc_apply(x, w_t, b_row, tm: int):
    n, k, h = x.shape
    kin = w_t.shape[0]
    out_dtype = x.dtype

    n_pad = _round_up(n, tm)
    if n_pad != n:
        x = jnp.pad(x, ((0, n_pad - n), (0, 0), (0, 0)))

    g = n_pad // tm

    cost = pl.CostEstimate(
        flops=2 * n_pad * kin * h,
        transcendentals=0,
        bytes_accessed=(n_pad * kin * 4 + 2 * kin * h * 4 + h * 4
                        + n_pad * h * 4),
    )

    out = pl.pallas_call(
        _make_fc_kernel(tm, k, h, g),
        out_shape=jax.ShapeDtypeStruct((n_pad, h), out_dtype),
        grid=(g,),
        in_specs=[pl.BlockSpec(memory_space=pltpu.MemorySpace.HBM),
                  pl.BlockSpec(memory_space=pltpu.MemorySpace.HBM),
                  pl.BlockSpec((1, h), lambda i: (0, 0))],
        out_specs=pl.BlockSpec((tm, h), lambda i: (i, 0)),
        scratch_shapes=[pltpu.VMEM((2, k, tm, h), jnp.float32),
                        pltpu.VMEM((kin, h), jnp.float32),
                        pltpu.SemaphoreType.DMA((2, k)),
                        pltpu.SemaphoreType.DMA((k,))],
        compiler_params=pltpu.CompilerParams(
            dimension_semantics=("parallel",),
            vmem_limit_bytes=60 * 1024 * 1024,
        ),
        cost_estimate=cost,
    )(x, w_t, b_row)

    if n_pad != n:
        out = out[:n]
    return out


def kernel(x, w_t, b_row):
    n = x.shape[0]
    tm = 1024 if n >= 2048 else max(8, _round_up(n // 2, 8))
    return _fc_apply(x, w_t, b_row, tm)


# speedup vs baseline: 1.0964x; 1.0021x over previous
import functools

import jax
import jax.numpy as jnp
from jax.experimental import pallas as pl
from jax.experimental.pallas import tpu as pltpu


def _round_up(a: int, b: int) -> int:
    return (a + b - 1) // b * b


def _make_fc_kernel(tm: int, k: int, h: int, g: int):
    g_half = g // 2 if g % 2 == 0 else g

    def _fc_kernel(x_hbm, w_hbm, b_ref, o_ref, xbuf, wbuf, xsems, wsems):
        i = pl.program_id(0)
        slot = jax.lax.rem(i, 2)

        def x_cp(step, sl, j):
            return pltpu.make_async_copy(
                x_hbm.at[pl.ds(step * tm, tm), j, :],
                xbuf.at[sl, j],
                xsems.at[sl, j])

        def w_cp(j):
            return pltpu.make_async_copy(
                w_hbm.at[pl.ds(j * h, h), :],
                wbuf.at[pl.ds(j * h, h), :],
                wsems.at[j])

        is_first = (i == 0) | (i == g_half)

        @pl.when(is_first)
        def _warmup():
            for j in range(k):
                x_cp(i, slot, j).start()
                w_cp(j).start()

        nxt_ok = (i + 1 < g) & (i + 1 != g_half)

        @pl.when(nxt_ok)
        def _prefetch():
            for j in range(k):
                x_cp(i + 1, 1 - slot, j).start()

        y = None
        for j in range(k):
            x_cp(i, slot, j).wait()

            @pl.when(is_first)
            def _wait_w():
                w_cp(j).wait()

            d = jnp.dot(xbuf[slot, j], wbuf[j * h:(j + 1) * h, :],
                        preferred_element_type=jnp.float32)
            y = d if y is None else y + d
        o_ref[...] = jnp.maximum(y + b_ref[...], 0.0).astype(o_ref.dtype)

    return _fc_kernel


@functools.partial(jax.jit, static_argnames=("tm",))
def _fc_apply(x, w_t, b_row, tm: int):
    n, k, h = x.shape
    kin = w_t.shape[0]
    out_dtype = x.dtype

    n_pad = _round_up(n, tm)
    if n_pad != n:
        x = jnp.pad(x, ((0, n_pad - n), (0, 0), (0, 0)))

    g = n_pad // tm

    cost = pl.CostEstimate(
        flops=2 * n_pad * kin * h,
        transcendentals=0,
        bytes_accessed=(n_pad * kin * 4 + 2 * kin * h * 4 + h * 4
                        + n_pad * h * 4),
    )

    out = pl.pallas_call(
        _make_fc_kernel(tm, k, h, g),
        out_shape=jax.ShapeDtypeStruct((n_pad, h), out_dtype),
        grid=(g,),
        in_specs=[pl.BlockSpec(memory_space=pltpu.MemorySpace.HBM),
                  pl.BlockSpec(memory_space=pltpu.MemorySpace.HBM),
                  pl.BlockSpec((1, h), lambda i: (0, 0))],
        out_specs=pl.BlockSpec((tm, h), lambda i: (i, 0)),
        scratch_shapes=[pltpu.VMEM((2, k, tm, h), jnp.float32),
                        pltpu.VMEM((kin, h), jnp.float32),
                        pltpu.SemaphoreType.DMA((2, k)),
                        pltpu.SemaphoreType.DMA((k,))],
        compiler_params=pltpu.CompilerParams(
            dimension_semantics=("parallel",),
            vmem_limit_bytes=60 * 1024 * 1024,
        ),
        cost_estimate=cost,
    )(x, w_t, b_row)

    if n_pad != n:
        out = out[:n]
    return out


def kernel(x, w_t, b_row):
    n = x.shape[0]
    tm = 512 if n >= 1024 else max(8, _round_up(n // 2, 8))
    return _fc_apply(x, w_t, b_row, tm)
